# baseline (speedup 1.0000x reference)
import functools

import jax
import jax.numpy as jnp
from jax import lax
from jax.experimental import pallas as pl
from jax.experimental.pallas import tpu as pltpu
from jax.experimental.pallas import tpu_sc as plsc

NUM_EMBEDDINGS = 8192
EMBEDDING_DIM = 256
BETA = 0.25


_BM = 1024
_BN = 2048
_NB = NUM_EMBEDDINGS // _BN


def _argmin_body(z_ref, w_ref, out_ref):
    zb = z_ref[...]
    best_val = None
    best_idx = None
    for nb in range(_NB):
        wb = w_ref[nb * _BN:(nb + 1) * _BN, :]
        prod = lax.dot_general(zb, wb, (((1,), (1,)), ((), ())),
                               preferred_element_type=jnp.float32)
        c = jnp.sum(wb * wb, axis=1)[None, :]
        e = prod - 0.5 * c
        row_max = jnp.max(e, axis=1, keepdims=True)
        idsf = lax.broadcasted_iota(jnp.int32, e.shape, 1).astype(jnp.float32)
        row_idx = jnp.min(jnp.where(e == row_max, idsf, jnp.float32(65536.0)),
                          axis=1, keepdims=True) + jnp.float32(nb * _BN)
        if nb == 0:
            best_val, best_idx = row_max, row_idx
        else:
            better = row_max > best_val
            best_val = jnp.where(better, row_max, best_val)
            best_idx = jnp.where(better, row_idx, best_idx)
    out_ref[...] = best_idx.astype(jnp.int32)


def _argmin_call(zf, weight):
    m_blocks = zf.shape[0] // _BM
    return pl.pallas_call(
        _argmin_body,
        grid=(m_blocks,),
        in_specs=[
            pl.BlockSpec((_BM, EMBEDDING_DIM), lambda m: (m, 0)),
            pl.BlockSpec((NUM_EMBEDDINGS, EMBEDDING_DIM), lambda m: (0, 0)),
        ],
        out_specs=pl.BlockSpec((_BM, 1), lambda m: (m, 0)),
        out_shape=jax.ShapeDtypeStruct((zf.shape[0], 1), jnp.int32),
        compiler_params=pltpu.CompilerParams(
            dimension_semantics=("arbitrary",)),
    )(zf, weight)


_NC = 2
_NS = 16
_NW = _NC * _NS
_TOKENS = 16384
_TPW = _TOKENS // _NW
_CH = 64
_NCH = _TPW // _CH
_IDX_W = 64
_SAMPLES = 16
_TOK_PER_SAMPLE = 1024


def _sc_body(w_hbm, idx_hbm, z_hbm, zq_hbm, loss_hbm, cnt_hbm,
             idx_v, rows0, rows1, zv0, zv1, pidx_v, pres_v, acc_v, cnt_v,
             g0, g1, zs0, zs1, o0, o1):
    wid = lax.axis_index("s") * _NC + lax.axis_index("c")
    pltpu.sync_copy(idx_hbm.at[pl.ds(wid * _NCH, _NCH)], idx_v)

    rows = (rows0, rows1)
    zv = (zv0, zv1)
    gsem = (g0, g1)
    zsem = (zs0, zs1)
    osem = (o0, o1)

    def tok0(ch):
        return wid * _TPW + ch * _CH

    gat = [None, None]
    zcp = [None, None]
    ocp = [None, None]
    gat[0] = pltpu.async_copy(w_hbm.at[idx_v.at[0]], rows[0], gsem[0])
    zcp[0] = pltpu.async_copy(z_hbm.at[pl.ds(tok0(0), _CH)], zv[0], zsem[0])

    acc = jnp.zeros((16,), jnp.float32)
    for ch in range(_NCH):
        b = ch % 2
        nb = (ch + 1) % 2
        if ch + 1 < _NCH:
            if ch >= 1:
                ocp[nb].wait()
            gat[nb] = pltpu.async_copy(w_hbm.at[idx_v.at[ch + 1]],
                                       rows[nb], gsem[nb])
            zcp[nb] = pltpu.async_copy(z_hbm.at[pl.ds(tok0(ch + 1), _CH)],
                                       zv[nb], zsem[nb])
        gat[b].wait()
        ocp[b] = pltpu.async_copy(rows[b], zq_hbm.at[pl.ds(tok0(ch), _CH)],
                                  osem[b])
        zcp[b].wait()
        rv, zr = rows[b], zv[b]

        def body(r, a, rv=rv, zr=zr):
            for cidx in range(EMBEDDING_DIM // 16):
                sl = pl.ds(cidx * 16, 16)
                dd = rv[r, sl] - zr[r, sl]
                a = a + dd * dd
            return a

        acc = lax.fori_loop(0, _CH, body, acc)
    ocp[0].wait()
    ocp[1].wait()

    acc_v[...] = acc
    pltpu.sync_copy(acc_v, loss_hbm.at[wid])

    @pl.when(wid < _SAMPLES)
    def _():
        srow = wid * (_TOK_PER_SAMPLE // _IDX_W)
        pltpu.sync_copy(idx_hbm.at[pl.ds(srow, _TOK_PER_SAMPLE // _IDX_W)],
                        pidx_v)

        def zero_body(i, _):
            pres_v[pl.ds(i * 16, 16)] = jnp.zeros((16,), jnp.float32)
            return 0

        lax.fori_loop(0, NUM_EMBEDDINGS // 16, zero_body, 0)
        ones = jnp.ones((16,), jnp.float32)
        for ra in range(_TOK_PER_SAMPLE // _IDX_W):
            for rb in range(_IDX_W // 16):
                iv = pidx_v[ra, pl.ds(rb * 16, 16)]
                plsc.store_scatter(pres_v, [iv], ones)

        def cnt_body(i, a):
            return a + pres_v[pl.ds(i * 16, 16)]

        cvec = lax.fori_loop(0, NUM_EMBEDDINGS // 16, cnt_body,
                             jnp.zeros((16,), jnp.float32))
        cnt_v[...] = cvec
        pltpu.sync_copy(cnt_v, cnt_hbm.at[wid])


@functools.partial(jax.jit, static_argnums=())
def _sc_call(weight, idx2, zf):
    kern = functools.partial(
        pl.kernel,
        out_type=[
            jax.ShapeDtypeStruct((_TOKENS, EMBEDDING_DIM), jnp.float32),
            jax.ShapeDtypeStruct((_NW, 16), jnp.float32),
            jax.ShapeDtypeStruct((_SAMPLES, 16), jnp.float32),
        ],
        mesh=plsc.VectorSubcoreMesh(core_axis_name="c", subcore_axis_name="s"),
        scratch_types=[
            pltpu.VMEM((_NCH, _IDX_W), jnp.int32),
            pltpu.VMEM((_CH, EMBEDDING_DIM), jnp.float32),
            pltpu.VMEM((_CH, EMBEDDING_DIM), jnp.float32),
            pltpu.VMEM((_CH, EMBEDDING_DIM), jnp.float32),
            pltpu.VMEM((_CH, EMBEDDING_DIM), jnp.float32),
            pltpu.VMEM((_TOK_PER_SAMPLE // _IDX_W, _IDX_W), jnp.int32),
            pltpu.VMEM((NUM_EMBEDDINGS,), jnp.float32),
            pltpu.VMEM((16,), jnp.float32),
            pltpu.VMEM((16,), jnp.float32),
            pltpu.SemaphoreType.DMA,
            pltpu.SemaphoreType.DMA,
            pltpu.SemaphoreType.DMA,
            pltpu.SemaphoreType.DMA,
            pltpu.SemaphoreType.DMA,
            pltpu.SemaphoreType.DMA,
        ],
        compiler_params=pltpu.CompilerParams(needs_layout_passes=False),
    )(_sc_body)
    return kern(weight, idx2, zf)


def kernel(z, weight):
    b, h, w, d = z.shape
    zf = z.reshape(-1, d)
    idx = _argmin_call(zf, weight)
    idx2 = idx.reshape(_TOKENS // _IDX_W, _IDX_W)
    zq_st, loss_parts, cnt_parts = _sc_call(weight, idx2, zf)
    mean_sq = jnp.sum(loss_parts) / jnp.float32(zf.size)
    loss = mean_sq + jnp.float32(BETA) * mean_sq
    diversity = jnp.sum(cnt_parts) / jnp.float32(_TOKENS)
    return (zq_st.reshape(b, h, w, d), idx.reshape(b, h, w), loss, diversity)


# speedup vs baseline: 1.0097x; 1.0097x over previous
import functools

import jax
import jax.numpy as jnp
from jax import lax
from jax.experimental import pallas as pl
from jax.experimental.pallas import tpu as pltpu
from jax.experimental.pallas import tpu_sc as plsc

NUM_EMBEDDINGS = 8192
EMBEDDING_DIM = 256
BETA = 0.25


_BM = 2048
_BN = 2048
_NB = NUM_EMBEDDINGS // _BN


def _argmin_body(z_ref, w_ref, out_ref, best_val, best_idx):
    n = pl.program_id(1)
    zb = z_ref[...]
    wb = w_ref[...]
    prod = lax.dot_general(zb, wb, (((1,), (1,)), ((), ())),
                           preferred_element_type=jnp.float32)
    c = jnp.sum(wb * wb, axis=1)[None, :]
    e = prod - 0.5 * c
    row_max = jnp.max(e, axis=1, keepdims=True)
    idsf = lax.broadcasted_iota(jnp.int32, e.shape, 1).astype(jnp.float32)
    row_idx = jnp.min(jnp.where(e == row_max, idsf, jnp.float32(65536.0)),
                      axis=1, keepdims=True) + jnp.float32(n * _BN)
    row_max = jnp.broadcast_to(row_max, (_BM, 128))
    row_idx = jnp.broadcast_to(row_idx, (_BM, 128))

    @pl.when(n == 0)
    def _():
        best_val[...] = row_max
        best_idx[...] = row_idx

    @pl.when(n > 0)
    def _():
        better = row_max > best_val[...]
        best_val[...] = jnp.where(better, row_max, best_val[...])
        best_idx[...] = jnp.where(better, row_idx, best_idx[...])

    @pl.when(n == _NB - 1)
    def _():
        out_ref[...] = best_idx[:, :1].astype(jnp.int32)


def _argmin_call(zf, weight):
    m_blocks = zf.shape[0] // _BM
    return pl.pallas_call(
        _argmin_body,
        grid=(m_blocks, _NB),
        in_specs=[
            pl.BlockSpec((_BM, EMBEDDING_DIM), lambda m, n: (m, 0)),
            pl.BlockSpec((_BN, EMBEDDING_DIM), lambda m, n: (n, 0)),
        ],
        out_specs=pl.BlockSpec((_BM, 1), lambda m, n: (m, 0)),
        out_shape=jax.ShapeDtypeStruct((zf.shape[0], 1), jnp.int32),
        scratch_shapes=[
            pltpu.VMEM((_BM, 128), jnp.float32),
            pltpu.VMEM((_BM, 128), jnp.float32),
        ],
        compiler_params=pltpu.CompilerParams(
            dimension_semantics=("arbitrary", "arbitrary")),
    )(zf, weight)


_NC = 2
_NS = 16
_NW = _NC * _NS
_TOKENS = 16384
_TPW = _TOKENS // _NW
_CH = 64
_NCH = _TPW // _CH
_IDX_W = 64
_SAMPLES = 16
_TOK_PER_SAMPLE = 1024


def _sc_body(w_hbm, idx_hbm, z_hbm, zq_hbm, loss_hbm, cnt_hbm,
             idx_v, rows0, rows1, zv0, zv1, pidx_v, pres_v, acc_v, cnt_v,
             g0, g1, zs0, zs1, o0, o1):
    wid = lax.axis_index("s") * _NC + lax.axis_index("c")
    pltpu.sync_copy(idx_hbm.at[pl.ds(wid * _NCH, _NCH)], idx_v)

    rows = (rows0, rows1)
    zv = (zv0, zv1)
    gsem = (g0, g1)
    zsem = (zs0, zs1)
    osem = (o0, o1)

    def tok0(ch):
        return wid * _TPW + ch * _CH

    gat = [None, None]
    zcp = [None, None]
    ocp = [None, None]
    gat[0] = pltpu.async_copy(w_hbm.at[idx_v.at[0]], rows[0], gsem[0])
    zcp[0] = pltpu.async_copy(z_hbm.at[pl.ds(tok0(0), _CH)], zv[0], zsem[0])

    acc = jnp.zeros((16,), jnp.float32)
    for ch in range(_NCH):
        b = ch % 2
        nb = (ch + 1) % 2
        if ch + 1 < _NCH:
            if ch >= 1:
                ocp[nb].wait()
            gat[nb] = pltpu.async_copy(w_hbm.at[idx_v.at[ch + 1]],
                                       rows[nb], gsem[nb])
            zcp[nb] = pltpu.async_copy(z_hbm.at[pl.ds(tok0(ch + 1), _CH)],
                                       zv[nb], zsem[nb])
        gat[b].wait()
        ocp[b] = pltpu.async_copy(rows[b], zq_hbm.at[pl.ds(tok0(ch), _CH)],
                                  osem[b])
        zcp[b].wait()
        rv, zr = rows[b], zv[b]

        def body(r, a, rv=rv, zr=zr):
            for cidx in range(EMBEDDING_DIM // 16):
                sl = pl.ds(cidx * 16, 16)
                dd = rv[r, sl] - zr[r, sl]
                a = a + dd * dd
            return a

        acc = lax.fori_loop(0, _CH, body, acc)
    ocp[0].wait()
    ocp[1].wait()

    acc_v[...] = acc
    pltpu.sync_copy(acc_v, loss_hbm.at[wid])

    @pl.when(wid < _SAMPLES)
    def _():
        srow = wid * (_TOK_PER_SAMPLE // _IDX_W)
        pltpu.sync_copy(idx_hbm.at[pl.ds(srow, _TOK_PER_SAMPLE // _IDX_W)],
                        pidx_v)

        def zero_body(i, _):
            pres_v[pl.ds(i * 16, 16)] = jnp.zeros((16,), jnp.float32)
            return 0

        lax.fori_loop(0, NUM_EMBEDDINGS // 16, zero_body, 0)
        ones = jnp.ones((16,), jnp.float32)
        for ra in range(_TOK_PER_SAMPLE // _IDX_W):
            for rb in range(_IDX_W // 16):
                iv = pidx_v[ra, pl.ds(rb * 16, 16)]
                plsc.store_scatter(pres_v, [iv], ones)

        def cnt_body(i, a):
            return a + pres_v[pl.ds(i * 16, 16)]

        cvec = lax.fori_loop(0, NUM_EMBEDDINGS // 16, cnt_body,
                             jnp.zeros((16,), jnp.float32))
        cnt_v[...] = cvec
        pltpu.sync_copy(cnt_v, cnt_hbm.at[wid])


@functools.partial(jax.jit, static_argnums=())
def _sc_call(weight, idx2, zf):
    kern = functools.partial(
        pl.kernel,
        out_type=[
            jax.ShapeDtypeStruct((_TOKENS, EMBEDDING_DIM), jnp.float32),
            jax.ShapeDtypeStruct((_NW, 16), jnp.float32),
            jax.ShapeDtypeStruct((_SAMPLES, 16), jnp.float32),
        ],
        mesh=plsc.VectorSubcoreMesh(core_axis_name="c", subcore_axis_name="s"),
        scratch_types=[
            pltpu.VMEM((_NCH, _IDX_W), jnp.int32),
            pltpu.VMEM((_CH, EMBEDDING_DIM), jnp.float32),
            pltpu.VMEM((_CH, EMBEDDING_DIM), jnp.float32),
            pltpu.VMEM((_CH, EMBEDDING_DIM), jnp.float32),
            pltpu.VMEM((_CH, EMBEDDING_DIM), jnp.float32),
            pltpu.VMEM((_TOK_PER_SAMPLE // _IDX_W, _IDX_W), jnp.int32),
            pltpu.VMEM((NUM_EMBEDDINGS,), jnp.float32),
            pltpu.VMEM((16,), jnp.float32),
            pltpu.VMEM((16,), jnp.float32),
            pltpu.SemaphoreType.DMA,
            pltpu.SemaphoreType.DMA,
            pltpu.SemaphoreType.DMA,
            pltpu.SemaphoreType.DMA,
            pltpu.SemaphoreType.DMA,
            pltpu.SemaphoreType.DMA,
        ],
        compiler_params=pltpu.CompilerParams(needs_layout_passes=False),
    )(_sc_body)
    return kern(weight, idx2, zf)


def kernel(z, weight):
    b, h, w, d = z.shape
    zf = z.reshape(-1, d)
    idx = _argmin_call(zf, weight)
    idx2 = idx.reshape(_TOKENS // _IDX_W, _IDX_W)
    zq_st, loss_parts, cnt_parts = _sc_call(weight, idx2, zf)
    mean_sq = jnp.sum(loss_parts) / jnp.float32(zf.size)
    loss = mean_sq + jnp.float32(BETA) * mean_sq
    diversity = jnp.sum(cnt_parts) / jnp.float32(_TOKENS)
    return (zq_st.reshape(b, h, w, d), idx.reshape(b, h, w), loss, diversity)
